# baseline (speedup 1.0000x reference)
import functools

import jax
import jax.numpy as jnp
from jax import lax
from jax.experimental import pallas as pl
from jax.experimental.pallas import tpu as pltpu
from jax.experimental.pallas import tpu_sc as plsc

BATCH = 16384
EMBED = 64
NC = 2
NS = 16
NW = NC * NS
B_PER_W = BATCH // NW
CHUNK = 128
N_CHUNKS = B_PER_W // CHUNK


def _gather_body(users_hbm, songs_hbm, ut_hbm, st_hbm, u_out, s_out,
                 idx_vu, idx_vs, sem_u, sem_s):
    wid = lax.axis_index("s") * NC + lax.axis_index("c")
    base = wid * B_PER_W
    pltpu.sync_copy(users_hbm.at[wid], idx_vu)
    pltpu.sync_copy(songs_hbm.at[wid], idx_vs)

    @pl.loop(0, B_PER_W // 16)
    def _(g):
        goff = g * 16
        vu = idx_vu[pl.ds(goff, 16)]
        vs = idx_vs[pl.ds(goff, 16)]
        for jj in range(16):
            pltpu.async_copy(ut_hbm.at[pl.ds(vu[jj], 1)],
                             u_out.at[pl.ds(base + goff + jj, 1)], sem_u)
            pltpu.async_copy(st_hbm.at[pl.ds(vs[jj], 1)],
                             s_out.at[pl.ds(base + goff + jj, 1)], sem_s)

    pltpu.make_async_copy(ut_hbm.at[pl.ds(0, B_PER_W)],
                          u_out.at[pl.ds(base, B_PER_W)], sem_u).wait()
    pltpu.make_async_copy(st_hbm.at[pl.ds(0, B_PER_W)],
                          s_out.at[pl.ds(base, B_PER_W)], sem_s).wait()


def _sc_gather(users, songs, user_table, song_table):
    mesh = plsc.VectorSubcoreMesh(core_axis_name="c", subcore_axis_name="s")
    f = pl.kernel(
        _gather_body,
        mesh=mesh,
        out_type=(
            jax.ShapeDtypeStruct((BATCH, EMBED), jnp.float32),
            jax.ShapeDtypeStruct((BATCH, EMBED), jnp.float32),
        ),
        scratch_types=[
            pltpu.VMEM((B_PER_W,), jnp.int32),
            pltpu.VMEM((B_PER_W,), jnp.int32),
            pltpu.SemaphoreType.DMA,
            pltpu.SemaphoreType.DMA,
        ],
    )
    return f(users.reshape(NW, B_PER_W), songs.reshape(NW, B_PER_W),
             user_table, song_table)


def _mlp_body(u_ref, s_ref, a_ref, b_ref, b1_ref, w2_ref, b2_ref, o_ref):
    dn = (((1,), (0,)), ((), ()))
    h = lax.dot_general(u_ref[...], a_ref[...], dn,
                        preferred_element_type=jnp.float32)
    h = h + lax.dot_general(s_ref[...], b_ref[...], dn,
                            preferred_element_type=jnp.float32)
    h = jnp.maximum(h + b1_ref[...], 0.0)
    o_ref[...] = jnp.sum(h * w2_ref[...], axis=1, keepdims=True) + b2_ref[...]


def _tc_mlp(u_rows, s_rows, w1a, w1b, b1, w2r, b2):
    bs = 2048
    grid = (BATCH // bs,)
    return pl.pallas_call(
        _mlp_body,
        grid=grid,
        in_specs=[
            pl.BlockSpec((bs, EMBED), lambda i: (i, 0)),
            pl.BlockSpec((bs, EMBED), lambda i: (i, 0)),
            pl.BlockSpec((EMBED, EMBED), lambda i: (0, 0)),
            pl.BlockSpec((EMBED, EMBED), lambda i: (0, 0)),
            pl.BlockSpec((1, EMBED), lambda i: (0, 0)),
            pl.BlockSpec((1, EMBED), lambda i: (0, 0)),
            pl.BlockSpec((1, 1), lambda i: (0, 0)),
        ],
        out_specs=pl.BlockSpec((bs, 1), lambda i: (i, 0)),
        out_shape=jax.ShapeDtypeStruct((BATCH, 1), jnp.float32),
    )(u_rows, s_rows, w1a, w1b, b1, w2r, b2)


def kernel(users, songs, user_table, song_table, W1, b1, W2, b2):
    users = users.astype(jnp.int32)
    songs = songs.astype(jnp.int32)
    u_rows, s_rows = _sc_gather(users, songs, user_table, song_table)
    return _tc_mlp(u_rows, s_rows,
                   W1[:EMBED], W1[EMBED:],
                   b1.reshape(1, EMBED),
                   W2.reshape(1, EMBED),
                   b2.reshape(1, 1))


# speedup vs baseline: 2.1502x; 2.1502x over previous
import functools

import jax
import jax.numpy as jnp
from jax import lax
from jax.experimental import pallas as pl
from jax.experimental.pallas import tpu as pltpu
from jax.experimental.pallas import tpu_sc as plsc

BATCH = 16384
EMBED = 64
NC = 2
NS = 16
NW = NC * NS
B_PER_W = BATCH // NW
CHUNK = 128
N_CHUNKS = B_PER_W // CHUNK


def _gather_body(users_hbm, songs_hbm, ut_hbm, st_hbm, u_out, s_out,
                 idx_vu, idx_vs, rows_u, rows_s, sem_u, sem_s):
    wid = lax.axis_index("s") * NC + lax.axis_index("c")
    base = wid * B_PER_W
    pltpu.sync_copy(users_hbm.at[wid], idx_vu)
    pltpu.sync_copy(songs_hbm.at[wid], idx_vs)

    for h in range(B_PER_W // CHUNK):
        off = h * CHUNK

        @pl.loop(0, CHUNK // 16)
        def _(g):
            goff = g * 16
            vu = idx_vu[pl.ds(off + goff, 16)]
            vs = idx_vs[pl.ds(off + goff, 16)]
            for jj in range(16):
                pltpu.async_copy(ut_hbm.at[pl.ds(vu[jj], 1)],
                                 rows_u.at[pl.ds(goff + jj, 1)], sem_u)
                pltpu.async_copy(st_hbm.at[pl.ds(vs[jj], 1)],
                                 rows_s.at[pl.ds(goff + jj, 1)], sem_s)

        pltpu.make_async_copy(ut_hbm.at[pl.ds(0, CHUNK)], rows_u, sem_u).wait()
        pltpu.make_async_copy(st_hbm.at[pl.ds(0, CHUNK)], rows_s, sem_s).wait()
        pltpu.sync_copy(rows_u, u_out.at[pl.ds(base + off, CHUNK)])
        pltpu.sync_copy(rows_s, s_out.at[pl.ds(base + off, CHUNK)])


def _sc_gather(users, songs, user_table, song_table):
    mesh = plsc.VectorSubcoreMesh(core_axis_name="c", subcore_axis_name="s")
    f = pl.kernel(
        _gather_body,
        mesh=mesh,
        out_type=(
            jax.ShapeDtypeStruct((BATCH, EMBED), jnp.float32),
            jax.ShapeDtypeStruct((BATCH, EMBED), jnp.float32),
        ),
        scratch_types=[
            pltpu.VMEM((B_PER_W,), jnp.int32),
            pltpu.VMEM((B_PER_W,), jnp.int32),
            pltpu.VMEM((CHUNK, EMBED), jnp.float32),
            pltpu.VMEM((CHUNK, EMBED), jnp.float32),
            pltpu.SemaphoreType.DMA,
            pltpu.SemaphoreType.DMA,
        ],
    )
    return f(users.reshape(NW, B_PER_W), songs.reshape(NW, B_PER_W),
             user_table, song_table)


def _mlp_body(u_ref, s_ref, a_ref, b_ref, b1_ref, w2_ref, b2_ref, o_ref):
    dn = (((1,), (0,)), ((), ()))
    h = lax.dot_general(u_ref[...], a_ref[...], dn,
                        preferred_element_type=jnp.float32)
    h = h + lax.dot_general(s_ref[...], b_ref[...], dn,
                            preferred_element_type=jnp.float32)
    h = jnp.maximum(h + b1_ref[...], 0.0)
    o_ref[...] = jnp.sum(h * w2_ref[...], axis=1, keepdims=True) + b2_ref[...]


def _tc_mlp(u_rows, s_rows, w1a, w1b, b1, w2r, b2):
    bs = 2048
    grid = (BATCH // bs,)
    return pl.pallas_call(
        _mlp_body,
        grid=grid,
        in_specs=[
            pl.BlockSpec((bs, EMBED), lambda i: (i, 0)),
            pl.BlockSpec((bs, EMBED), lambda i: (i, 0)),
            pl.BlockSpec((EMBED, EMBED), lambda i: (0, 0)),
            pl.BlockSpec((EMBED, EMBED), lambda i: (0, 0)),
            pl.BlockSpec((1, EMBED), lambda i: (0, 0)),
            pl.BlockSpec((1, EMBED), lambda i: (0, 0)),
            pl.BlockSpec((1, 1), lambda i: (0, 0)),
        ],
        out_specs=pl.BlockSpec((bs, 1), lambda i: (i, 0)),
        out_shape=jax.ShapeDtypeStruct((BATCH, 1), jnp.float32),
    )(u_rows, s_rows, w1a, w1b, b1, w2r, b2)


def kernel(users, songs, user_table, song_table, W1, b1, W2, b2):
    users = users.astype(jnp.int32)
    songs = songs.astype(jnp.int32)
    u_rows, s_rows = _sc_gather(users, songs, user_table, song_table)
    return _tc_mlp(u_rows, s_rows,
                   W1[:EMBED], W1[EMBED:],
                   b1.reshape(1, EMBED),
                   W2.reshape(1, EMBED),
                   b2.reshape(1, 1))
